# baseline (speedup 1.0000x reference)
import functools

import jax
import jax.numpy as jnp
from jax import lax
from jax.experimental import pallas as pl
from jax.experimental.pallas import tpu as pltpu
from jax.experimental.pallas import tpu_sc as plsc

N = 10000
E = 320000
EA = E + N
ALPHA = 1.0

NC = 2
NS = 16
NW = NC * NS
L = 16
CH = 48
RPT = 224
EAP = NW * RPT * CH
NPAD = 10240
NPT = NPAD // NS
HW = 128


def _sc_gat_kernel(d, h_hbm, pk_hbm, src_hbm, dst_hbm,
                   num_hbm, den_hbm,
                   pk_v, den_v, sring, dring,
                   gbuf0, gbuf1, sbuf0, sbuf1,
                   out_acc, isem, gsem, ssem):
    cid = lax.axis_index("c")
    sid = lax.axis_index("s")
    nt = d // L
    w = cid * NS + sid
    ebase = w * RPT * CH

    gbufs = (gbuf0, gbuf1)
    sbufs = (sbuf0, sbuf1)

    pltpu.sync_copy(pk_hbm, pk_v)
    HMASK = jnp.int32(-65536)

    zero16f = jnp.zeros((L,), jnp.float32)
    zero16i = jnp.zeros((L,), jnp.int32)

    def _zero_den(i, c):
        den_v[0, pl.ds(i * L, L)] = zero16f
        return c
    lax.fori_loop(0, NPAD // L, _zero_den, 0)

    def _zero_sbuf(i, c):
        sbuf0[i // (HW // L), pl.ds((i % (HW // L)) * L, L)] = zero16f
        sbuf1[i // (HW // L), pl.ds((i % (HW // L)) * L, L)] = zero16f
        return c
    lax.fori_loop(0, CH * (HW // L), _zero_sbuf, 0)
    for t in range(NPT // CH):
        pltpu.sync_copy(sbuf0, out_acc.at[pl.ds(sid * NPT + t * CH, CH)])
    plsc.subcore_barrier()

    def _idx_descr(k, slot):
        off = ebase + jnp.minimum(k, RPT - 1) * CH
        return (
            pltpu.make_async_copy(src_hbm.at[pl.ds(off, CH)],
                                  sring.at[slot], isem.at[slot]),
            pltpu.make_async_copy(dst_hbm.at[pl.ds(off, CH)],
                                  dring.at[slot], isem.at[slot]),
        )

    def _idx_start(k, slot):
        for d_ in _idx_descr(k, slot):
            d_.start()

    def _idx_wait(k, slot):
        for d_ in _idx_descr(k, slot):
            d_.wait()

    def _gather_descr(islot, gslot):
        return pltpu.make_async_copy(h_hbm.at[sring.at[islot]],
                                     gbufs[gslot], gsem.at[gslot])

    def _scatter_descr(islot, gslot):
        return pltpu.make_async_copy(sbufs[gslot],
                                     out_acc.at[dring.at[islot]],
                                     ssem.at[gslot])

    def _edge_body(k, a):
        _idx_wait(k + 1, (a + 1) % 4)
        _gather_descr((a + 1) % 4, (a + 1) % 2).start()
        _gather_descr(a % 4, a % 2).wait()
        _scatter_descr(a % 4, a % 2).wait()
        _idx_start(k + 2, (a + 2) % 4)
        gbuf = gbufs[a % 2]
        sbuf = sbufs[a % 2]
        for jj in range(CH // L):
            sv = sring[a, pl.ds(jj * L, L)]
            dv = dring[a, pl.ds(jj * L, L)]
            ts = plsc.load_gather(pk_v, [sv])
            td = plsc.load_gather(pk_v, [dv])
            al = (plsc.bitcast(lax.shift_left(ts, 16), jnp.float32)
                  + plsc.bitcast(jnp.bitwise_and(td, HMASK), jnp.float32))
            al = jnp.maximum(al, 0.2 * al)
            ex = jnp.where(dv < N, jnp.exp(al), 0.0)
            plsc.addupdate_scatter(den_v, [zero16i, dv], ex)
            for r in range(L):
                cr = ex[r]
                for t in range(nt):
                    sbuf[jj * L + r, pl.ds(t * L, L)] = (
                        gbuf[jj * L + r, pl.ds(t * L, L)] * cr)
        _scatter_descr(a % 4, a % 2).start(add=True)

    _idx_start(0, 0)
    _idx_start(1, 1)
    for j in range(CH // L):
        dring[2, pl.ds(j * L, L)] = zero16i
        dring[3, pl.ds(j * L, L)] = zero16i
    pltpu.make_async_copy(sbuf0, out_acc.at[dring.at[2]],
                          ssem.at[0]).start(add=True)
    pltpu.make_async_copy(sbuf1, out_acc.at[dring.at[3]],
                          ssem.at[1]).start(add=True)
    _idx_wait(0, 0)
    _gather_descr(0, 0).start()

    def _edge_quad(kk, c):
        for a in range(4):
            _edge_body(4 * kk + a, a)
        return c
    lax.fori_loop(0, RPT // 4, _edge_quad, 0)

    _idx_wait(RPT + 1, (RPT + 1) % 4)
    _gather_descr(RPT % 4, RPT % 2).wait()
    _scatter_descr(RPT % 4, 0).wait()
    _scatter_descr((RPT + 1) % 4, 1).wait()
    plsc.subcore_barrier()

    pltpu.sync_copy(den_v, den_hbm.at[cid].at[sid])
    rows = pl.ds(sid * NPT, NPT)
    pltpu.sync_copy(out_acc.at[rows], num_hbm.at[cid].at[rows])


@functools.lru_cache(maxsize=None)
def _make_sc_gat(d):
    mesh = plsc.VectorSubcoreMesh(core_axis_name="c", subcore_axis_name="s")
    return pl.kernel(
        functools.partial(_sc_gat_kernel, d),
        out_type=[
            jax.ShapeDtypeStruct((NC, NPAD, HW), jnp.float32),
            jax.ShapeDtypeStruct((NC, NS, 1, NPAD), jnp.float32),
        ],
        mesh=mesh,
        compiler_params=pltpu.CompilerParams(needs_layout_passes=False),
        scratch_types=[
            pltpu.VMEM((NPAD,), jnp.int32),
            pltpu.VMEM((1, NPAD), jnp.float32),
            pltpu.VMEM((4, CH), jnp.int32),
            pltpu.VMEM((4, CH), jnp.int32),
            pltpu.VMEM((CH, HW), jnp.float32),
            pltpu.VMEM((CH, HW), jnp.float32),
            pltpu.VMEM((CH, HW), jnp.float32),
            pltpu.VMEM((CH, HW), jnp.float32),
            pltpu.VMEM_SHARED((NPAD, HW), jnp.float32),
            pltpu.SemaphoreType.DMA((4,)),
            pltpu.SemaphoreType.DMA((2,)),
            pltpu.SemaphoreType.DMA((2,)),
        ],
    )


_BN = 1280


def _pad_h(h):
    if h.shape[1] < HW:
        h = jnp.concatenate(
            [h, jnp.zeros((h.shape[0], HW - h.shape[1]), jnp.float32)], axis=1)
    return h


def _matmuls(v, w_ref, a_ref, h_ref, al_ref):
    h = jax.lax.dot_general(v, w_ref[...], (((1,), (0,)), ((), ())),
                            preferred_element_type=jnp.float32)
    h_ref[...] = _pad_h(h)
    al_ref[...] = jax.lax.dot_general(h, a_ref[...], (((1,), (0,)), ((), ())),
                                      preferred_element_type=jnp.float32)


def _tc_pre_kernel(x_ref, w_ref, a_ref, h_ref, al_ref):
    x = x_ref[...]
    nrm = jnp.sqrt(jnp.sum(x * x, axis=1, keepdims=True))
    x = x / jnp.maximum(nrm, 1e-12)
    _matmuls(x, w_ref, a_ref, h_ref, al_ref)


def _gat_out(num_ref, den_ref, b_ref, dprev):
    den = jnp.sum(den_ref[...], axis=(0, 1, 2))
    inv = 1.0 / (den + 1e-16)
    vfull = (num_ref[0] + num_ref[1]) * inv[:, None]
    return vfull[:, :dprev] + b_ref[...][None, :]


def _tc_mid_kernel(mode, dprev, num_ref, den_ref, b_ref, w_ref, a_ref,
                   *out_refs):
    v = _gat_out(num_ref, den_ref, b_ref, dprev)
    if mode == "relu":
        h_ref, al_ref = out_refs
        v = jnp.maximum(v, 0.0)
    else:
        z_ref, h_ref, al_ref = out_refs
        nrm = jnp.sqrt(jnp.sum(v * v, axis=1, keepdims=True))
        v = v / jnp.maximum(nrm, 1e-12)
        z_ref[...] = v
    _matmuls(v, w_ref, a_ref, h_ref, al_ref)


def _tc_fin_kernel(num_ref, den_ref, b_ref, z_ref, c_ref, xh_ref, q_ref):
    xh_ref[...] = _gat_out(num_ref, den_ref, b_ref, 128)
    z = z_ref[...]
    c = c_ref[...]
    zn = jnp.sum(z * z, axis=1, keepdims=True)
    cn = jnp.sum(c * c, axis=1, keepdims=True)
    dist = zn + cn.T - 2.0 * jax.lax.dot_general(
        z, c, (((1,), (1,)), ((), ())), preferred_element_type=jnp.float32)
    q = 1.0 / (1.0 + dist / ALPHA) + 1e-07
    q_ref[...] = q / jnp.sum(q, axis=1, keepdims=True)


def _num_den_specs():
    return [
        pl.BlockSpec((NC, _BN, HW), lambda i: (0, i, 0)),
        pl.BlockSpec((NC, NS, 1, _BN), lambda i: (0, 0, 0, i)),
    ]


def _tc_pre(x, w, a):
    di, do = w.shape
    return pl.pallas_call(
        _tc_pre_kernel,
        grid=(NPAD // _BN,),
        in_specs=[
            pl.BlockSpec((_BN, di), lambda i: (i, 0)),
            pl.BlockSpec((di, do), lambda i: (0, 0)),
            pl.BlockSpec((do, 8), lambda i: (0, 0)),
        ],
        out_specs=[
            pl.BlockSpec((_BN, HW), lambda i: (i, 0)),
            pl.BlockSpec((_BN, 8), lambda i: (i, 0)),
        ],
        out_shape=[
            jax.ShapeDtypeStruct((NPAD, HW), jnp.float32),
            jax.ShapeDtypeStruct((NPAD, 8), jnp.float32),
        ],
    )(x, w, a)


def _tc_mid(mode, dprev, num, den, b, w, a):
    di, do = w.shape
    out_specs = [
        pl.BlockSpec((_BN, HW), lambda i: (i, 0)),
        pl.BlockSpec((_BN, 8), lambda i: (i, 0)),
    ]
    out_shape = [
        jax.ShapeDtypeStruct((NPAD, HW), jnp.float32),
        jax.ShapeDtypeStruct((NPAD, 8), jnp.float32),
    ]
    if mode == "l2norm":
        out_specs = [pl.BlockSpec((_BN, di), lambda i: (i, 0))] + out_specs
        out_shape = [jax.ShapeDtypeStruct((NPAD, di), jnp.float32)] + out_shape
    return pl.pallas_call(
        functools.partial(_tc_mid_kernel, mode, dprev),
        grid=(NPAD // _BN,),
        in_specs=_num_den_specs() + [
            pl.BlockSpec((di,), lambda i: (0,)),
            pl.BlockSpec((di, do), lambda i: (0, 0)),
            pl.BlockSpec((do, 8), lambda i: (0, 0)),
        ],
        out_specs=out_specs,
        out_shape=out_shape,
    )(num, den, b, w, a)


def _tc_fin(num, den, b, z, cluster):
    return pl.pallas_call(
        _tc_fin_kernel,
        grid=(NPAD // _BN,),
        in_specs=_num_den_specs() + [
            pl.BlockSpec((128,), lambda i: (0,)),
            pl.BlockSpec((_BN, 64), lambda i: (i, 0)),
            pl.BlockSpec((16, 64), lambda i: (0, 0)),
        ],
        out_specs=[
            pl.BlockSpec((_BN, 128), lambda i: (i, 0)),
            pl.BlockSpec((_BN, 16), lambda i: (i, 0)),
        ],
        out_shape=[
            jax.ShapeDtypeStruct((NPAD, 128), jnp.float32),
            jax.ShapeDtypeStruct((NPAD, 16), jnp.float32),
        ],
    )(num, den, b, z, cluster)


def _amat(a_s, a_d):
    do = a_s.shape[0]
    return jnp.concatenate(
        [a_s[:, None], a_d[:, None], jnp.zeros((do, 6), jnp.float32)], axis=1)


def _sc_gat(d, h, al, srcp, dstp):
    pk = jax.lax.bitcast_convert_type(
        al[:, :2].astype(jnp.bfloat16), jnp.int32)
    return _make_sc_gat(d)(h, pk, srcp, dstp)


def kernel(x, edge_index, W1, as1, ad1, b1, W2, as2, ad2, b2, W3, as3, ad3,
           b3, W4, as4, ad4, b4, cluster):
    loop = jnp.arange(N, dtype=edge_index.dtype)
    srcp = jnp.concatenate(
        [edge_index[0], loop, jnp.zeros((EAP - EA,), edge_index.dtype)])
    dstp = jnp.concatenate(
        [edge_index[1], loop, jnp.full((EAP - EA,), N, edge_index.dtype)])
    xp = jnp.concatenate([x, jnp.zeros((NPAD - N, 128), jnp.float32)])

    h1, al1 = _tc_pre(xp, W1, _amat(as1, ad1))
    n1, d1 = _sc_gat(128, h1, al1, srcp, dstp)
    h2, al2 = _tc_mid("relu", 128, n1, d1, b1, W2, _amat(as2, ad2))
    n2, d2 = _sc_gat(64, h2, al2, srcp, dstp)
    z, h3, al3 = _tc_mid("l2norm", 64, n2, d2, b2, W3, _amat(as3, ad3))
    n3, d3 = _sc_gat(128, h3, al3, srcp, dstp)
    h4, al4 = _tc_mid("relu", 128, n3, d3, b3, W4, _amat(as4, ad4))
    n4, d4 = _sc_gat(128, h4, al4, srcp, dstp)
    x_hat, q = _tc_fin(n4, d4, b4, z, cluster)
    return (z[:N], x_hat[:N], q[:N])


# speedup vs baseline: 1.2392x; 1.2392x over previous
import functools

import jax
import jax.numpy as jnp
from jax import lax
from jax.experimental import pallas as pl
from jax.experimental.pallas import tpu as pltpu
from jax.experimental.pallas import tpu_sc as plsc

N = 10000
E = 320000
EA = E + N
ALPHA = 1.0

NC = 2
NS = 16
NW = NC * NS
L = 16
CH = 32
RPT = 336
EAP = NW * RPT * CH
NPAD = 10240
NPT = NPAD // NS
HW = 128


def _sc_gat_kernel(d, h_hbm, asrc_hbm, adst_hbm, src_hbm, dst_hbm,
                   num_hbm, den_hbm,
                   asrc_v, adst_v, den_v, sring, dring,
                   gbuf0, gbuf1, sbuf0, sbuf1,
                   out_acc, isem, gsem, ssem):
    cid = lax.axis_index("c")
    sid = lax.axis_index("s")
    nt = d // L
    w = cid * NS + sid
    ebase = w * RPT * CH

    gbufs = (gbuf0, gbuf1)
    sbufs = (sbuf0, sbuf1)

    pltpu.sync_copy(asrc_hbm, asrc_v)
    pltpu.sync_copy(adst_hbm, adst_v)

    zero16f = jnp.zeros((L,), jnp.float32)
    zero16i = jnp.zeros((L,), jnp.int32)

    def _zero_den(i, c):
        den_v[0, pl.ds(i * L, L)] = zero16f
        return c
    lax.fori_loop(0, NPAD // L, _zero_den, 0)

    def _zero_sbuf(i, c):
        sbuf0[i // (HW // L), pl.ds((i % (HW // L)) * L, L)] = zero16f
        sbuf1[i // (HW // L), pl.ds((i % (HW // L)) * L, L)] = zero16f
        return c
    lax.fori_loop(0, CH * (HW // L), _zero_sbuf, 0)
    for t in range(NPT // CH):
        pltpu.sync_copy(sbuf0, out_acc.at[pl.ds(sid * NPT + t * CH, CH)])
    plsc.subcore_barrier()

    def _idx_descr(k, slot):
        off = ebase + jnp.minimum(k, RPT - 1) * CH
        return (
            pltpu.make_async_copy(src_hbm.at[pl.ds(off, CH)],
                                  sring.at[slot], isem.at[slot]),
            pltpu.make_async_copy(dst_hbm.at[pl.ds(off, CH)],
                                  dring.at[slot], isem.at[slot]),
        )

    def _idx_start(k, slot):
        for d_ in _idx_descr(k, slot):
            d_.start()

    def _idx_wait(k, slot):
        for d_ in _idx_descr(k, slot):
            d_.wait()

    def _gather_descr(islot, gslot):
        return pltpu.make_async_copy(h_hbm.at[sring.at[islot]],
                                     gbufs[gslot], gsem.at[gslot])

    def _scatter_descr(islot, gslot):
        return pltpu.make_async_copy(sbufs[gslot],
                                     out_acc.at[dring.at[islot]],
                                     ssem.at[gslot])

    def _edge_body(k, a):
        _idx_wait(k + 1, (a + 1) % 4)
        _gather_descr((a + 1) % 4, (a + 1) % 2).start()
        _gather_descr(a % 4, a % 2).wait()
        _scatter_descr(a % 4, a % 2).wait()
        _idx_start(k + 2, (a + 2) % 4)
        gbuf = gbufs[a % 2]
        sbuf = sbufs[a % 2]
        for jj in range(CH // L):
            sv = sring[a, pl.ds(jj * L, L)]
            dv = dring[a, pl.ds(jj * L, L)]
            al = (plsc.load_gather(asrc_v, [sv])
                  + plsc.load_gather(adst_v, [dv]))
            al = jnp.maximum(al, 0.2 * al)
            ex = jnp.where(dv < N, jnp.exp(al), 0.0)
            plsc.addupdate_scatter(den_v, [zero16i, dv], ex)
            for r in range(L):
                cr = ex[r]
                for t in range(nt):
                    sbuf[jj * L + r, pl.ds(t * L, L)] = (
                        gbuf[jj * L + r, pl.ds(t * L, L)] * cr)
        _scatter_descr(a % 4, a % 2).start(add=True)

    _idx_start(0, 0)
    _idx_start(1, 1)
    for j in range(CH // L):
        dring[2, pl.ds(j * L, L)] = zero16i
        dring[3, pl.ds(j * L, L)] = zero16i
    pltpu.make_async_copy(sbuf0, out_acc.at[dring.at[2]],
                          ssem.at[0]).start(add=True)
    pltpu.make_async_copy(sbuf1, out_acc.at[dring.at[3]],
                          ssem.at[1]).start(add=True)
    _idx_wait(0, 0)
    _gather_descr(0, 0).start()

    def _edge_quad(kk, c):
        for a in range(4):
            _edge_body(4 * kk + a, a)
        return c
    lax.fori_loop(0, RPT // 4, _edge_quad, 0)

    _idx_wait(RPT + 1, (RPT + 1) % 4)
    _gather_descr(RPT % 4, RPT % 2).wait()
    _scatter_descr(RPT % 4, 0).wait()
    _scatter_descr((RPT + 1) % 4, 1).wait()
    plsc.subcore_barrier()

    pltpu.sync_copy(den_v, den_hbm.at[cid].at[sid])
    rows = pl.ds(sid * NPT, NPT)
    pltpu.sync_copy(out_acc.at[rows], num_hbm.at[cid].at[rows])


@functools.lru_cache(maxsize=None)
def _make_sc_gat(d):
    mesh = plsc.VectorSubcoreMesh(core_axis_name="c", subcore_axis_name="s")
    return pl.kernel(
        functools.partial(_sc_gat_kernel, d),
        out_type=[
            jax.ShapeDtypeStruct((NC, NPAD, HW), jnp.float32),
            jax.ShapeDtypeStruct((NC, NS, 1, NPAD), jnp.float32),
        ],
        mesh=mesh,
        compiler_params=pltpu.CompilerParams(needs_layout_passes=False),
        scratch_types=[
            pltpu.VMEM((NPAD,), jnp.float32),
            pltpu.VMEM((NPAD,), jnp.float32),
            pltpu.VMEM((1, NPAD), jnp.float32),
            pltpu.VMEM((4, CH), jnp.int32),
            pltpu.VMEM((4, CH), jnp.int32),
            pltpu.VMEM((CH, HW), jnp.float32),
            pltpu.VMEM((CH, HW), jnp.float32),
            pltpu.VMEM((CH, HW), jnp.float32),
            pltpu.VMEM((CH, HW), jnp.float32),
            pltpu.VMEM_SHARED((NPAD, HW), jnp.float32),
            pltpu.SemaphoreType.DMA((4,)),
            pltpu.SemaphoreType.DMA((2,)),
            pltpu.SemaphoreType.DMA((2,)),
        ],
    )


_BN = 1280


def _pad_h(h):
    if h.shape[1] < HW:
        h = jnp.concatenate(
            [h, jnp.zeros((h.shape[0], HW - h.shape[1]), jnp.float32)], axis=1)
    return h


def _matmuls(v, w_ref, a_ref, h_ref, al_ref):
    h = jax.lax.dot_general(v, w_ref[...], (((1,), (0,)), ((), ())),
                            preferred_element_type=jnp.float32)
    h_ref[...] = _pad_h(h)
    al_ref[...] = jax.lax.dot_general(h, a_ref[...], (((1,), (0,)), ((), ())),
                                      preferred_element_type=jnp.float32)


def _tc_pre_kernel(x_ref, w_ref, a_ref, h_ref, al_ref):
    x = x_ref[...]
    nrm = jnp.sqrt(jnp.sum(x * x, axis=1, keepdims=True))
    x = x / jnp.maximum(nrm, 1e-12)
    _matmuls(x, w_ref, a_ref, h_ref, al_ref)


def _gat_out(num_ref, den_ref, b_ref, dprev):
    den = jnp.sum(den_ref[...], axis=(0, 1, 2))
    inv = 1.0 / (den + 1e-16)
    vfull = (num_ref[0] + num_ref[1]) * inv[:, None]
    return vfull[:, :dprev] + b_ref[...][None, :]


def _tc_mid_kernel(mode, dprev, num_ref, den_ref, b_ref, w_ref, a_ref,
                   *out_refs):
    v = _gat_out(num_ref, den_ref, b_ref, dprev)
    if mode == "relu":
        h_ref, al_ref = out_refs
        v = jnp.maximum(v, 0.0)
    else:
        z_ref, h_ref, al_ref = out_refs
        nrm = jnp.sqrt(jnp.sum(v * v, axis=1, keepdims=True))
        v = v / jnp.maximum(nrm, 1e-12)
        z_ref[...] = v
    _matmuls(v, w_ref, a_ref, h_ref, al_ref)


def _tc_fin_kernel(num_ref, den_ref, b_ref, z_ref, c_ref, xh_ref, q_ref):
    xh_ref[...] = _gat_out(num_ref, den_ref, b_ref, 128)
    z = z_ref[...]
    c = c_ref[...]
    zn = jnp.sum(z * z, axis=1, keepdims=True)
    cn = jnp.sum(c * c, axis=1, keepdims=True)
    dist = zn + cn.T - 2.0 * jax.lax.dot_general(
        z, c, (((1,), (1,)), ((), ())), preferred_element_type=jnp.float32)
    q = 1.0 / (1.0 + dist / ALPHA) + 1e-07
    q_ref[...] = q / jnp.sum(q, axis=1, keepdims=True)


def _num_den_specs():
    return [
        pl.BlockSpec((NC, _BN, HW), lambda i: (0, i, 0)),
        pl.BlockSpec((NC, NS, 1, _BN), lambda i: (0, 0, 0, i)),
    ]


def _tc_pre(x, w, a):
    di, do = w.shape
    return pl.pallas_call(
        _tc_pre_kernel,
        grid=(NPAD // _BN,),
        in_specs=[
            pl.BlockSpec((_BN, di), lambda i: (i, 0)),
            pl.BlockSpec((di, do), lambda i: (0, 0)),
            pl.BlockSpec((do, 8), lambda i: (0, 0)),
        ],
        out_specs=[
            pl.BlockSpec((_BN, HW), lambda i: (i, 0)),
            pl.BlockSpec((_BN, 8), lambda i: (i, 0)),
        ],
        out_shape=[
            jax.ShapeDtypeStruct((NPAD, HW), jnp.float32),
            jax.ShapeDtypeStruct((NPAD, 8), jnp.float32),
        ],
    )(x, w, a)


def _tc_mid(mode, dprev, num, den, b, w, a):
    di, do = w.shape
    out_specs = [
        pl.BlockSpec((_BN, HW), lambda i: (i, 0)),
        pl.BlockSpec((_BN, 8), lambda i: (i, 0)),
    ]
    out_shape = [
        jax.ShapeDtypeStruct((NPAD, HW), jnp.float32),
        jax.ShapeDtypeStruct((NPAD, 8), jnp.float32),
    ]
    if mode == "l2norm":
        out_specs = [pl.BlockSpec((_BN, di), lambda i: (i, 0))] + out_specs
        out_shape = [jax.ShapeDtypeStruct((NPAD, di), jnp.float32)] + out_shape
    return pl.pallas_call(
        functools.partial(_tc_mid_kernel, mode, dprev),
        grid=(NPAD // _BN,),
        in_specs=_num_den_specs() + [
            pl.BlockSpec((di,), lambda i: (0,)),
            pl.BlockSpec((di, do), lambda i: (0, 0)),
            pl.BlockSpec((do, 8), lambda i: (0, 0)),
        ],
        out_specs=out_specs,
        out_shape=out_shape,
    )(num, den, b, w, a)


def _tc_fin(num, den, b, z, cluster):
    return pl.pallas_call(
        _tc_fin_kernel,
        grid=(NPAD // _BN,),
        in_specs=_num_den_specs() + [
            pl.BlockSpec((128,), lambda i: (0,)),
            pl.BlockSpec((_BN, 64), lambda i: (i, 0)),
            pl.BlockSpec((16, 64), lambda i: (0, 0)),
        ],
        out_specs=[
            pl.BlockSpec((_BN, 128), lambda i: (i, 0)),
            pl.BlockSpec((_BN, 16), lambda i: (i, 0)),
        ],
        out_shape=[
            jax.ShapeDtypeStruct((NPAD, 128), jnp.float32),
            jax.ShapeDtypeStruct((NPAD, 16), jnp.float32),
        ],
    )(num, den, b, z, cluster)


def _amat(a_s, a_d):
    do = a_s.shape[0]
    return jnp.concatenate(
        [a_s[:, None], a_d[:, None], jnp.zeros((do, 6), jnp.float32)], axis=1)


def _sc_gat(d, h, al, srcp, dstp):
    return _make_sc_gat(d)(h, al[:, 0], al[:, 1], srcp, dstp)


def kernel(x, edge_index, W1, as1, ad1, b1, W2, as2, ad2, b2, W3, as3, ad3,
           b3, W4, as4, ad4, b4, cluster):
    loop = jnp.arange(N, dtype=edge_index.dtype)
    srcp = jnp.concatenate(
        [edge_index[0], loop, jnp.zeros((EAP - EA,), edge_index.dtype)])
    dstp = jnp.concatenate(
        [edge_index[1], loop, jnp.full((EAP - EA,), N, edge_index.dtype)])
    xp = jnp.concatenate([x, jnp.zeros((NPAD - N, 128), jnp.float32)])

    h1, al1 = _tc_pre(xp, W1, _amat(as1, ad1))
    n1, d1 = _sc_gat(128, h1, al1, srcp, dstp)
    h2, al2 = _tc_mid("relu", 128, n1, d1, b1, W2, _amat(as2, ad2))
    n2, d2 = _sc_gat(64, h2, al2, srcp, dstp)
    z, h3, al3 = _tc_mid("l2norm", 64, n2, d2, b2, W3, _amat(as3, ad3))
    n3, d3 = _sc_gat(128, h3, al3, srcp, dstp)
    h4, al4 = _tc_mid("relu", 128, n3, d3, b3, W4, _amat(as4, ad4))
    n4, d4 = _sc_gat(128, h4, al4, srcp, dstp)
    x_hat, q = _tc_fin(n4, d4, b4, z, cluster)
    return (z[:N], x_hat[:N], q[:N])
